# baseline (speedup 1.0000x reference)
import functools

import jax
import jax.numpy as jnp
from jax import lax
from jax.experimental import pallas as pl
from jax.experimental.pallas import tpu as pltpu
from jax.experimental.pallas import tpu_sc as plsc

B = 16384
DIM = 64
MD = 32
PD = 32
LW = 128
NU = 100000
HALF = NU // 2
QUART = NU // 4

NC = 2
NS = 16
NW = NC * NS
BPW = B // NW
CH = 64
NCH = BPW // CH

RB = 2000
RB4 = 1000
RG = HALF // RB


def _repack_body(ua, ub, ia, ib, ma, mb, mc, md_, pa, pb, pc, pd_,
                 ue2, ie2, me2, pe2):
    ue2[...] = jnp.concatenate([ua[...], ub[...]], axis=1)
    ie2[...] = jnp.concatenate([ia[...], ib[...]], axis=1)
    me2[...] = jnp.concatenate([ma[...], mb[...], mc[...], md_[...]], axis=1)
    pe2[...] = jnp.concatenate([pa[...], pb[...], pc[...], pd_[...]], axis=1)


def _repack(user_emb, item_emb, emb_manu, emb_part):
    half = lambda off: pl.BlockSpec((RB, DIM), lambda i, o=off: (o + i, 0))
    quart = lambda off: pl.BlockSpec((RB4, MD), lambda i, o=off: (o + i, 0))
    return pl.pallas_call(
        _repack_body,
        grid=(RG,),
        in_specs=[
            half(0), half(RG), half(0), half(RG),
            quart(0), quart(RG), quart(2 * RG), quart(3 * RG),
            quart(0), quart(RG), quart(2 * RG), quart(3 * RG),
        ],
        out_specs=[
            pl.BlockSpec((RB, LW), lambda i: (i, 0)),
            pl.BlockSpec((RB, LW), lambda i: (i, 0)),
            pl.BlockSpec((RB4, LW), lambda i: (i, 0)),
            pl.BlockSpec((RB4, LW), lambda i: (i, 0)),
        ],
        out_shape=[
            jax.ShapeDtypeStruct((HALF, LW), jnp.float32),
            jax.ShapeDtypeStruct((HALF, LW), jnp.float32),
            jax.ShapeDtypeStruct((QUART, LW), jnp.float32),
            jax.ShapeDtypeStruct((QUART, LW), jnp.float32),
        ],
    )(user_emb, user_emb, item_emb, item_emb,
      emb_manu, emb_manu, emb_manu, emb_manu,
      emb_part, emb_part, emb_part, emb_part)


def _sc_gather_body(u_idx, i_idx, m_idx, p_idx,
                    user_emb, item_emb, emb_manu, emb_part,
                    out_u, out_i, out_m, out_p,
                    vu_idx, vi_idx, vm_idx, vp_idx,
                    ru0, ri0, rm0, rp0, ru1, ri1, rm1, rp1,
                    s0, s1, s2, s3):
    wid = lax.axis_index("c") * NS + lax.axis_index("s")
    base = wid * BPW

    pltpu.sync_copy(u_idx.at[wid], vu_idx)
    pltpu.sync_copy(i_idx.at[wid], vi_idx)
    pltpu.sync_copy(m_idx.at[wid], vm_idx)
    pltpu.sync_copy(p_idx.at[wid], vp_idx)

    bufs = ((ru0, ri0, rm0, rp0), (ru1, ri1, rm1, rp1))
    tabs = (user_emb, item_emb, emb_manu, emb_part)
    outs = (out_u, out_i, out_m, out_p)
    idxs = (vu_idx, vi_idx, vm_idx, vp_idx)
    sems = (s0, s1, s2, s3)

    def fire(j):
        bset = bufs[j % 2]
        return [pltpu.async_copy(tabs[t].at[idxs[t].at[j]], bset[t], sems[t])
                for t in range(4)]

    pending = fire(0)
    for j in range(NCH):
        nxt = fire(j + 1) if j + 1 < NCH else None
        for c in pending:
            c.wait()
        bset = bufs[j % 2]
        off = base + j * CH
        for t in range(4):
            pltpu.sync_copy(bset[t], outs[t].at[pl.ds(off, CH)])
        pending = nxt


def _make_sc_gather():
    return functools.partial(
        pl.kernel,
        mesh=plsc.VectorSubcoreMesh(core_axis_name="c", subcore_axis_name="s"),
        out_type=[
            jax.ShapeDtypeStruct((B, LW), jnp.float32),
            jax.ShapeDtypeStruct((B, LW), jnp.float32),
            jax.ShapeDtypeStruct((B, LW), jnp.float32),
            jax.ShapeDtypeStruct((B, LW), jnp.float32),
        ],
        scratch_types=(
            [pltpu.VMEM((NCH, CH), jnp.int32) for _ in range(4)]
            + [pltpu.VMEM((CH, LW), jnp.float32) for _ in range(8)]
            + [pltpu.SemaphoreType.DMA for _ in range(4)]
        ),
    )(_sc_gather_body)


def _mlp_body(year, uid, iid, mid, pid, u128, ic128, m128, p128,
              Wy1, by1, Wy2, by2, Wp, bp, Wm1, bm1, Wm2, bm2, Who, bho,
              out):
    f32 = jnp.float32
    relu = lambda a: jnp.maximum(a, 0.0)

    u = jnp.where(uid[...] < HALF, u128[:, 0:64], u128[:, 64:128])
    ic = jnp.where(iid[...] < HALF, ic128[:, 0:64], ic128[:, 64:128])

    def pick4(idx, g):
        hi = idx >= HALF
        odd = (idx - jnp.where(hi, HALF, 0)) >= QUART
        a = jnp.where(hi, g[:, 64:96], g[:, 0:32])
        b = jnp.where(hi, g[:, 96:128], g[:, 32:64])
        return jnp.where(odd, b, a)

    m = pick4(mid[...], m128)
    p = pick4(pid[...], p128)

    y1 = relu(year[...] * Wy1[...] + by1[...])
    y = relu(jnp.dot(y1, Wy2[...], preferred_element_type=f32) + by2[...])
    cin = jnp.concatenate([y, m, p], axis=1)
    cont = relu(jnp.dot(cin, Wp[...], preferred_element_type=f32) + bp[...])
    x = jnp.concatenate([u, ic, cont], axis=1)
    h1 = relu(jnp.dot(x, Wm1[...], preferred_element_type=f32) + bm1[...])
    h = relu(jnp.dot(h1, Wm2[...], preferred_element_type=f32) + bm2[...])
    out[...] = jnp.dot(h, Who[...], preferred_element_type=f32) + bho[...]


def kernel(users, items, item_year, item_manu, item_part,
           user_emb, item_emb, emb_manu, emb_part,
           W_y1, b_y1, W_y2, b_y2, W_proj, b_proj,
           W_m1, b_m1, W_m2, b_m2, W_he, b_he, W_hi, b_hi, W_g, b_g):
    i32 = jnp.int32
    users = users.astype(i32)
    items = items.astype(i32)
    item_manu = item_manu.astype(i32)
    item_part = item_part.astype(i32)

    ue2 = jnp.concatenate([user_emb[:HALF], user_emb[HALF:]], axis=1)
    ie2 = jnp.concatenate([item_emb[:HALF], item_emb[HALF:]], axis=1)
    me2 = jnp.concatenate([emb_manu[0:QUART], emb_manu[QUART:2 * QUART],
                           emb_manu[2 * QUART:3 * QUART], emb_manu[3 * QUART:]],
                          axis=1)
    pe2 = jnp.concatenate([emb_part[0:QUART], emb_part[QUART:2 * QUART],
                           emb_part[2 * QUART:3 * QUART], emb_part[3 * QUART:]],
                          axis=1)

    u_idx = jnp.where(users < HALF, users, users - HALF).reshape(NW, NCH, CH)
    i_idx = jnp.where(items < HALF, items, items - HALF).reshape(NW, NCH, CH)
    m_idx = (item_manu % QUART).reshape(NW, NCH, CH)
    p_idx = (item_part % QUART).reshape(NW, NCH, CH)

    u_g, ic_g, m_g, p_g = _make_sc_gather()(
        u_idx, i_idx, m_idx, p_idx, ue2, ie2, me2, pe2)

    Who = jnp.concatenate([W_he, W_hi], axis=1)
    bho = jnp.concatenate([b_he, b_hi]).reshape(1, 2)

    bs = 2048
    grid = (B // bs,)
    row_spec = lambda d: pl.BlockSpec((bs, d), lambda gi: (gi, 0))
    full = lambda a: pl.BlockSpec(a.shape, lambda gi: (0,) * a.ndim)

    out2 = pl.pallas_call(
        _mlp_body,
        grid=grid,
        in_specs=[
            row_spec(1), row_spec(1), row_spec(1), row_spec(1), row_spec(1),
            row_spec(LW), row_spec(LW), row_spec(LW), row_spec(LW),
            full(W_y1), full(b_y1.reshape(1, -1)),
            full(W_y2), full(b_y2.reshape(1, -1)),
            full(W_proj), full(b_proj.reshape(1, -1)),
            full(W_m1), full(b_m1.reshape(1, -1)),
            full(W_m2), full(b_m2.reshape(1, -1)),
            full(Who), full(bho),
        ],
        out_specs=pl.BlockSpec((bs, 2), lambda gi: (gi, 0)),
        out_shape=jax.ShapeDtypeStruct((B, 2), jnp.float32),
    )(item_year, users.reshape(B, 1), items.reshape(B, 1),
      item_manu.reshape(B, 1), item_part.reshape(B, 1),
      u_g, ic_g, m_g, p_g,
      W_y1, b_y1.reshape(1, -1), W_y2, b_y2.reshape(1, -1),
      W_proj, b_proj.reshape(1, -1), W_m1, b_m1.reshape(1, -1),
      W_m2, b_m2.reshape(1, -1), Who, bho)

    return (out2[:, 0:1], out2[:, 1:2])


# speedup vs baseline: 1.4975x; 1.4975x over previous
import functools

import jax
import jax.numpy as jnp
from jax import lax
from jax.experimental import pallas as pl
from jax.experimental.pallas import tpu as pltpu
from jax.experimental.pallas import tpu_sc as plsc

B = 16384
DIM = 64
MD = 32
PD = 32

NC = 2
NS = 16
NW = NC * NS
BPW = B // NW
CH = 128
NCH = BPW // CH


def _sc_gather_body(u_idx, i_idx, m_idx, p_idx,
                    user_emb, item_emb, emb_manu, emb_part,
                    out_u, out_i, out_m, out_p,
                    vu_idx, vi_idx, vm_idx, vp_idx,
                    ru, ri, rm, rp,
                    s0, s1, s2, s3):
    wid = lax.axis_index("c") * NS + lax.axis_index("s")
    base = wid * BPW

    pltpu.sync_copy(u_idx.at[wid], vu_idx)
    pltpu.sync_copy(i_idx.at[wid], vi_idx)
    pltpu.sync_copy(m_idx.at[wid], vm_idx)
    pltpu.sync_copy(p_idx.at[wid], vp_idx)

    copies = []
    for j in range(NCH):
        copies.append(pltpu.async_copy(
            user_emb.at[vu_idx.at[j]], ru.at[pl.ds(j * CH, CH)], s0))
        copies.append(pltpu.async_copy(
            item_emb.at[vi_idx.at[j]], ri.at[pl.ds(j * CH, CH)], s1))
        copies.append(pltpu.async_copy(
            emb_manu.at[vm_idx.at[j]], rm.at[pl.ds(j * CH, CH)], s2))
        copies.append(pltpu.async_copy(
            emb_part.at[vp_idx.at[j]], rp.at[pl.ds(j * CH, CH)], s3))
    for c in copies:
        c.wait()

    pltpu.sync_copy(ru, out_u.at[pl.ds(base, BPW)])
    pltpu.sync_copy(ri, out_i.at[pl.ds(base, BPW)])
    pltpu.sync_copy(rm, out_m.at[pl.ds(base, BPW)])
    pltpu.sync_copy(rp, out_p.at[pl.ds(base, BPW)])


def _make_sc_gather():
    return functools.partial(
        pl.kernel,
        mesh=plsc.VectorSubcoreMesh(core_axis_name="c", subcore_axis_name="s"),
        compiler_params=pltpu.CompilerParams(use_tc_tiling_on_sc=False),
        out_type=[
            jax.ShapeDtypeStruct((B, DIM), jnp.float32),
            jax.ShapeDtypeStruct((B, DIM), jnp.float32),
            jax.ShapeDtypeStruct((B, MD), jnp.float32),
            jax.ShapeDtypeStruct((B, PD), jnp.float32),
        ],
        scratch_types=[
            pltpu.VMEM((NCH, CH), jnp.int32),
            pltpu.VMEM((NCH, CH), jnp.int32),
            pltpu.VMEM((NCH, CH), jnp.int32),
            pltpu.VMEM((NCH, CH), jnp.int32),
            pltpu.VMEM((BPW, DIM), jnp.float32),
            pltpu.VMEM((BPW, DIM), jnp.float32),
            pltpu.VMEM((BPW, MD), jnp.float32),
            pltpu.VMEM((BPW, PD), jnp.float32),
            pltpu.SemaphoreType.DMA,
            pltpu.SemaphoreType.DMA,
            pltpu.SemaphoreType.DMA,
            pltpu.SemaphoreType.DMA,
        ],
    )(_sc_gather_body)


def _mlp_body(year, u, ic, m, p,
              Wy1, by1, Wy2, by2, Wp, bp, Wm1, bm1, Wm2, bm2,
              Whe, bhe, Whi, bhi,
              out_e, out_i):
    f32 = jnp.float32
    relu = lambda a: jnp.maximum(a, 0.0)
    y1 = relu(year[...] * Wy1[...] + by1[...].reshape(1, -1))
    y = relu(jnp.dot(y1, Wy2[...], preferred_element_type=f32)
             + by2[...].reshape(1, -1))
    cin = jnp.concatenate([y, m[...], p[...]], axis=1)
    cont = relu(jnp.dot(cin, Wp[...], preferred_element_type=f32)
                + bp[...].reshape(1, -1))
    x = jnp.concatenate([u[...], ic[...], cont], axis=1)
    h1 = relu(jnp.dot(x, Wm1[...], preferred_element_type=f32)
              + bm1[...].reshape(1, -1))
    h = relu(jnp.dot(h1, Wm2[...], preferred_element_type=f32)
             + bm2[...].reshape(1, -1))
    out_e[...] = jnp.dot(h, Whe[...], preferred_element_type=f32) + bhe[...]
    out_i[...] = jnp.dot(h, Whi[...], preferred_element_type=f32) + bhi[...]


def kernel(users, items, item_year, item_manu, item_part,
           user_emb, item_emb, emb_manu, emb_part,
           W_y1, b_y1, W_y2, b_y2, W_proj, b_proj,
           W_m1, b_m1, W_m2, b_m2, W_he, b_he, W_hi, b_hi, W_g, b_g):
    i32 = jnp.int32
    u_idx = users.astype(i32).reshape(NW, NCH, CH)
    i_idx = items.astype(i32).reshape(NW, NCH, CH)
    m_idx = item_manu.astype(i32).reshape(NW, NCH, CH)
    p_idx = item_part.astype(i32).reshape(NW, NCH, CH)

    u_g, ic_g, m_g, p_g = _make_sc_gather()(
        u_idx, i_idx, m_idx, p_idx, user_emb, item_emb, emb_manu, emb_part)

    bs = 4096
    grid = (B // bs,)
    row_spec = lambda d: pl.BlockSpec((bs, d), lambda gi: (gi, 0))
    full = lambda a: pl.BlockSpec(a.shape, lambda gi: (0,) * a.ndim)

    out_e, out_i = pl.pallas_call(
        _mlp_body,
        grid=grid,
        in_specs=[
            row_spec(1), row_spec(DIM), row_spec(DIM), row_spec(MD), row_spec(PD),
            full(W_y1), full(b_y1), full(W_y2), full(b_y2),
            full(W_proj), full(b_proj), full(W_m1), full(b_m1),
            full(W_m2), full(b_m2),
            full(W_he), full(b_he), full(W_hi), full(b_hi),
        ],
        out_specs=[pl.BlockSpec((bs, 1), lambda gi: (gi, 0)),
                   pl.BlockSpec((bs, 1), lambda gi: (gi, 0))],
        out_shape=[jax.ShapeDtypeStruct((B, 1), jnp.float32),
                   jax.ShapeDtypeStruct((B, 1), jnp.float32)],
    )(item_year, u_g, ic_g, m_g, p_g,
      W_y1, b_y1, W_y2, b_y2, W_proj, b_proj,
      W_m1, b_m1, W_m2, b_m2, W_he, b_he, W_hi, b_hi)

    return (out_e, out_i)


# speedup vs baseline: 1.5715x; 1.0494x over previous
import functools

import jax
import jax.numpy as jnp
from jax import lax
from jax.experimental import pallas as pl
from jax.experimental.pallas import tpu as pltpu
from jax.experimental.pallas import tpu_sc as plsc

B = 16384
DIM = 64
MD = 32
PD = 32
LW = 128

NC = 2
NS = 16
NW = NC * NS
BPW = B // NW
CH = 64
NCH = BPW // CH


def _sc_gather_body(u_idx, i_idx, m_idx, p_idx,
                    user_emb, item_emb, emb_manu, emb_part,
                    out_u, out_i, out_m, out_p,
                    vu_idx, vi_idx, vm_idx, vp_idx,
                    ru0, ri0, rm0, rp0, ru1, ri1, rm1, rp1,
                    s0, s1, s2, s3):
    wid = lax.axis_index("c") * NS + lax.axis_index("s")
    base = wid * BPW

    pltpu.sync_copy(u_idx.at[wid], vu_idx)
    pltpu.sync_copy(i_idx.at[wid], vi_idx)
    pltpu.sync_copy(m_idx.at[wid], vm_idx)
    pltpu.sync_copy(p_idx.at[wid], vp_idx)

    bufs = ((ru0, ri0, rm0, rp0), (ru1, ri1, rm1, rp1))
    tabs = (user_emb, item_emb, emb_manu, emb_part)
    outs = (out_u, out_i, out_m, out_p)
    idxs = (vu_idx, vi_idx, vm_idx, vp_idx)
    sems = (s0, s1, s2, s3)

    def fire(j):
        bset = bufs[j % 2]
        return [pltpu.async_copy(tabs[t].at[idxs[t].at[j]], bset[t], sems[t])
                for t in range(4)]

    pending = fire(0)
    for j in range(NCH):
        nxt = fire(j + 1) if j + 1 < NCH else None
        for c in pending:
            c.wait()
        bset = bufs[j % 2]
        off = base + j * CH
        for t in range(4):
            pltpu.sync_copy(bset[t], outs[t].at[pl.ds(off, CH)])
        pending = nxt


def _make_sc_gather():
    return functools.partial(
        pl.kernel,
        mesh=plsc.VectorSubcoreMesh(core_axis_name="c", subcore_axis_name="s"),
        out_type=[
            jax.ShapeDtypeStruct((B, LW), jnp.float32),
            jax.ShapeDtypeStruct((B, LW), jnp.float32),
            jax.ShapeDtypeStruct((B, LW), jnp.float32),
            jax.ShapeDtypeStruct((B, LW), jnp.float32),
        ],
        scratch_types=(
            [pltpu.VMEM((NCH, CH), jnp.int32) for _ in range(4)]
            + [pltpu.VMEM((CH, LW), jnp.float32) for _ in range(8)]
            + [pltpu.SemaphoreType.DMA for _ in range(4)]
        ),
    )(_sc_gather_body)


def _mlp_body(year, u128, ic128, m128, p128,
              Wy1, by1, Wy2, by2, Wp, bp, Wm1, bm1, Wm2, bm2,
              Whe, bhe, Whi, bhi,
              out_e, out_i):
    f32 = jnp.float32
    relu = lambda a: jnp.maximum(a, 0.0)
    u = u128[:, 0:DIM]
    ic = ic128[:, 0:DIM]
    m = m128[:, 0:MD]
    p = p128[:, 0:PD]
    y1 = relu(year[...] * Wy1[...] + by1[...].reshape(1, -1))
    y = relu(jnp.dot(y1, Wy2[...], preferred_element_type=f32)
             + by2[...].reshape(1, -1))
    cin = jnp.concatenate([y, m, p], axis=1)
    cont = relu(jnp.dot(cin, Wp[...], preferred_element_type=f32)
                + bp[...].reshape(1, -1))
    x = jnp.concatenate([u, ic, cont], axis=1)
    h1 = relu(jnp.dot(x, Wm1[...], preferred_element_type=f32)
              + bm1[...].reshape(1, -1))
    h = relu(jnp.dot(h1, Wm2[...], preferred_element_type=f32)
             + bm2[...].reshape(1, -1))
    out_e[...] = jnp.dot(h, Whe[...], preferred_element_type=f32) + bhe[...]
    out_i[...] = jnp.dot(h, Whi[...], preferred_element_type=f32) + bhi[...]


def kernel(users, items, item_year, item_manu, item_part,
           user_emb, item_emb, emb_manu, emb_part,
           W_y1, b_y1, W_y2, b_y2, W_proj, b_proj,
           W_m1, b_m1, W_m2, b_m2, W_he, b_he, W_hi, b_hi, W_g, b_g):
    i32 = jnp.int32
    u_idx = users.astype(i32).reshape(NW, NCH, CH)
    i_idx = items.astype(i32).reshape(NW, NCH, CH)
    m_idx = item_manu.astype(i32).reshape(NW, NCH, CH)
    p_idx = item_part.astype(i32).reshape(NW, NCH, CH)

    uep = jnp.pad(user_emb, ((0, 0), (0, LW - DIM)))
    iep = jnp.pad(item_emb, ((0, 0), (0, LW - DIM)))
    mep = jnp.pad(emb_manu, ((0, 0), (0, LW - MD)))
    pep = jnp.pad(emb_part, ((0, 0), (0, LW - PD)))

    u_g, ic_g, m_g, p_g = _make_sc_gather()(
        u_idx, i_idx, m_idx, p_idx, uep, iep, mep, pep)

    bs = 4096
    grid = (B // bs,)
    row_spec = lambda d: pl.BlockSpec((bs, d), lambda gi: (gi, 0))
    full = lambda a: pl.BlockSpec(a.shape, lambda gi: (0,) * a.ndim)

    out_e, out_i = pl.pallas_call(
        _mlp_body,
        grid=grid,
        in_specs=[
            row_spec(1), row_spec(LW), row_spec(LW), row_spec(LW), row_spec(LW),
            full(W_y1), full(b_y1), full(W_y2), full(b_y2),
            full(W_proj), full(b_proj), full(W_m1), full(b_m1),
            full(W_m2), full(b_m2),
            full(W_he), full(b_he), full(W_hi), full(b_hi),
        ],
        out_specs=[pl.BlockSpec((bs, 1), lambda gi: (gi, 0)),
                   pl.BlockSpec((bs, 1), lambda gi: (gi, 0))],
        out_shape=[jax.ShapeDtypeStruct((B, 1), jnp.float32),
                   jax.ShapeDtypeStruct((B, 1), jnp.float32)],
    )(item_year, u_g, ic_g, m_g, p_g,
      W_y1, b_y1, W_y2, b_y2, W_proj, b_proj,
      W_m1, b_m1, W_m2, b_m2, W_he, b_he, W_hi, b_hi)

    return (out_e, out_i)
